# baseline (speedup 1.0000x reference)
import math

import jax
import jax.numpy as jnp
from jax.experimental import pallas as pl
from jax.experimental.pallas import tpu as pltpu

_B, _H, _W = 2, 224, 224
_TR = 224
_NT = _H // _TR
_R = 4
_SPAN = 2 * _R + 3


def _param_kernel(inp_ref, w1_ref, b1_ref, w2_ref, b2_ref, out_ref):
    t = pl.program_id(1)
    y0 = t * _TR
    slab = inp_ref[0, :, pl.ds(y0, _TR + 8), :]
    cols = []
    for ci in range(3):
        for ky in range(3):
            for kx in range(3):
                cols.append(slab[ci, ky:ky + _TR, kx:kx + _W])
    col = jnp.stack(cols)
    feat = jax.lax.dot_general(
        w1_ref[...], col, (((1,), (0,)), ((), ())),
        preferred_element_type=jnp.float32)
    feat = jnp.maximum(feat + b1_ref[...][:, :, None], 0.0)
    pred = jax.lax.dot_general(
        w2_ref[...], feat, (((1,), (0,)), ((), ())),
        preferred_element_type=jnp.float32)
    pred = pred + b2_ref[...][:, :, None]

    theta = jax.nn.sigmoid(pred[3]) * (2.0 * math.pi)
    ct = jnp.cos(theta)
    st = jnp.sin(theta)
    sx = (jax.nn.sigmoid(pred[4]) * 0.5 + 1e-6) * (_W * 0.5)
    sy = (jax.nn.sigmoid(pred[5]) * 0.5 + 1e-6) * (_H * 0.5)
    sx2 = sx * sx
    sy2 = sy * sy
    a = ct * ct * sx2 + st * st * sy2
    bcov = ct * st * (sx2 - sy2)
    c = st * st * sx2 + ct * ct * sy2
    det = a * c - bcov * bcov + 1e-12
    inv_det = 1.0 / det

    out_ref[0, 0] = pred[0]
    out_ref[0, 1] = pred[1]
    out_ref[0, 2] = pred[2]
    out_ref[0, 3] = jnp.tanh(pred[6]) - 0.5
    out_ref[0, 4] = jnp.tanh(pred[7]) - 0.5
    out_ref[0, 5] = -0.5 * c * inv_det
    out_ref[0, 6] = bcov * inv_det
    out_ref[0, 7] = -0.5 * a * inv_det


def _raster_kernel(p_ref, out_ref):
    t = pl.program_id(1)
    y0 = t * _TR
    blk = p_ref[0, :, pl.ds(y0, _TR + 16), :]
    partials = []
    for dw in range(_SPAN):
        dwr = dw - _R
        a = float(0.5 - dwr)
        slab = blk[:, :, dw:dw + _W]
        r0s, r1s, r2s = slab[0], slab[1], slab[2]
        qxs, qys = slab[3], slab[4]
        nAs, nBs, nCs = slab[5], slab[6], slab[7]
        if dwr == -_R:
            mxs = jnp.floor(qxs) >= -0.5
        elif dwr == -_R + 1:
            mxs = jnp.floor(qxs) >= -1.5
        elif dwr == _R + 1:
            mxs = jnp.floor(qxs) <= -0.5
        elif dwr == _R + 2:
            mxs = jnp.floor(qxs) <= -1.5
        else:
            mxs = None
        gys = jnp.floor(qys)
        acc0 = jnp.zeros((_TR, _W), jnp.float32)
        acc1 = jnp.zeros((_TR, _W), jnp.float32)
        acc2 = jnp.zeros((_TR, _W), jnp.float32)
        for dh in range(_SPAN):
            dhr = dh - _R
            b = float(0.5 - dhr)
            qx = qxs[dh:dh + _TR]
            qy = qys[dh:dh + _TR]
            nA = nAs[dh:dh + _TR]
            nB = nBs[dh:dh + _TR]
            nC = nCs[dh:dh + _TR]
            fx = a - qx
            fy = b - qy
            power = fx * (nA * fx + nB * fy) + nC * (fy * fy)
            alpha = jnp.exp(jnp.minimum(power, 0.0))
            if dhr == -_R:
                mys = gys[dh:dh + _TR] >= -0.5
            elif dhr == -_R + 1:
                mys = gys[dh:dh + _TR] >= -1.5
            elif dhr == _R + 1:
                mys = gys[dh:dh + _TR] <= -0.5
            elif dhr == _R + 2:
                mys = gys[dh:dh + _TR] <= -1.5
            else:
                mys = None
            if mxs is not None and mys is not None:
                m = mxs[dh:dh + _TR] & mys
            elif mxs is not None:
                m = mxs[dh:dh + _TR]
            else:
                m = mys
            wgt = alpha if m is None else jnp.where(m, alpha, 0.0)
            acc0 = acc0 + wgt * r0s[dh:dh + _TR]
            acc1 = acc1 + wgt * r1s[dh:dh + _TR]
            acc2 = acc2 + wgt * r2s[dh:dh + _TR]
        partials.append((acc0, acc1, acc2))
    while len(partials) > 1:
        nxt = []
        for i in range(0, len(partials) - 1, 2):
            p, q = partials[i], partials[i + 1]
            nxt.append((p[0] + q[0], p[1] + q[1], p[2] + q[2]))
        if len(partials) % 2:
            nxt.append(partials[-1])
        partials = nxt
    tot0, tot1, tot2 = partials[0]
    out_ref[0, 0] = jnp.clip(tot0, 0.0, 1.0)
    out_ref[0, 1] = jnp.clip(tot1, 0.0, 1.0)
    out_ref[0, 2] = jnp.clip(tot2, 0.0, 1.0)


_PARALLEL = pltpu.CompilerParams(dimension_semantics=("parallel", "parallel"))


def kernel(inp, enc_w, enc_b, head_w, head_b):
    w1 = enc_w.reshape(64, 27).astype(jnp.float32)
    w2 = head_w.reshape(8, 64).astype(jnp.float32)
    b1 = enc_b.reshape(64, 1).astype(jnp.float32)
    b2 = head_b.reshape(8, 1).astype(jnp.float32)
    inp_p = jnp.pad(inp.astype(jnp.float32), ((0, 0), (0, 0), (1, 7), (1, 1)))

    params = pl.pallas_call(
        _param_kernel,
        grid=(_B, _NT),
        in_specs=[
            pl.BlockSpec((1, 3, _H + 8, _W + 2), lambda b, t: (b, 0, 0, 0)),
            pl.BlockSpec((64, 27), lambda b, t: (0, 0)),
            pl.BlockSpec((64, 1), lambda b, t: (0, 0)),
            pl.BlockSpec((8, 64), lambda b, t: (0, 0)),
            pl.BlockSpec((8, 1), lambda b, t: (0, 0)),
        ],
        out_specs=pl.BlockSpec((1, 8, _TR, _W), lambda b, t: (b, 0, t, 0)),
        out_shape=jax.ShapeDtypeStruct((_B, 8, _H, _W), jnp.float32),
        compiler_params=_PARALLEL,
    )(inp_p, w1, b1, w2, b2)

    params_p = jnp.pad(params, ((0, 0), (0, 0), (_R, _R + 8), (_R, _R + 2)))

    img = pl.pallas_call(
        _raster_kernel,
        grid=(_B, _NT),
        in_specs=[
            pl.BlockSpec((1, 8, _H + 16, _W + _SPAN - 1),
                         lambda b, t: (b, 0, 0, 0)),
        ],
        out_specs=pl.BlockSpec((1, 3, _TR, _W), lambda b, t: (b, 0, t, 0)),
        out_shape=jax.ShapeDtypeStruct((_B, 3, _H, _W), jnp.float32),
        compiler_params=_PARALLEL,
    )(params_p)
    return img


# speedup vs baseline: 1.1651x; 1.1651x over previous
import math

import jax
import jax.numpy as jnp
from jax.experimental import pallas as pl
from jax.experimental.pallas import tpu as pltpu

_B, _H, _W = 2, 224, 224
_TR = 112
_NT = _H // _TR
_R = 4
_SPAN = 2 * _R + 3


def _param_kernel(inp_ref, w1_ref, b1_ref, w2_ref, b2_ref, out_ref):
    t = pl.program_id(1)
    y0 = t * _TR
    slab = inp_ref[0, :, pl.ds(y0, _TR + 8), :]
    cols = []
    for ci in range(3):
        for ky in range(3):
            for kx in range(3):
                cols.append(slab[ci, ky:ky + _TR, kx:kx + _W])
    col = jnp.stack(cols)
    feat = jax.lax.dot_general(
        w1_ref[...], col, (((1,), (0,)), ((), ())),
        preferred_element_type=jnp.float32)
    feat = jnp.maximum(feat + b1_ref[...][:, :, None], 0.0)
    pred = jax.lax.dot_general(
        w2_ref[...], feat, (((1,), (0,)), ((), ())),
        preferred_element_type=jnp.float32)
    pred = pred + b2_ref[...][:, :, None]

    theta = jax.nn.sigmoid(pred[3]) * (2.0 * math.pi)
    ct = jnp.cos(theta)
    st = jnp.sin(theta)
    sx = (jax.nn.sigmoid(pred[4]) * 0.5 + 1e-6) * (_W * 0.5)
    sy = (jax.nn.sigmoid(pred[5]) * 0.5 + 1e-6) * (_H * 0.5)
    sx2 = sx * sx
    sy2 = sy * sy
    a = ct * ct * sx2 + st * st * sy2
    bcov = ct * st * (sx2 - sy2)
    c = st * st * sx2 + ct * ct * sy2
    det = a * c - bcov * bcov + 1e-12
    inv_det = 1.0 / det

    out_ref[0, 0] = pred[0]
    out_ref[0, 1] = pred[1]
    out_ref[0, 2] = pred[2]
    out_ref[0, 3] = jnp.tanh(pred[6]) - 0.5
    out_ref[0, 4] = jnp.tanh(pred[7]) - 0.5
    out_ref[0, 5] = -0.5 * c * inv_det
    out_ref[0, 6] = bcov * inv_det
    out_ref[0, 7] = -0.5 * a * inv_det


def _raster_kernel(p_ref, out_ref):
    t = pl.program_id(1)
    y0 = t * _TR
    blk = p_ref[0, :, pl.ds(y0, _TR + 16), :]
    partials = []
    for dw in range(_SPAN):
        dwr = dw - _R
        a = float(0.5 - dwr)
        slab = blk[:, :, dw:dw + _W]
        r0s, r1s, r2s = slab[0], slab[1], slab[2]
        qxs, qys = slab[3], slab[4]
        nAs, nBs, nCs = slab[5], slab[6], slab[7]
        if dwr == -_R:
            mxs = jnp.floor(qxs) >= -0.5
        elif dwr == -_R + 1:
            mxs = jnp.floor(qxs) >= -1.5
        elif dwr == _R + 1:
            mxs = jnp.floor(qxs) <= -0.5
        elif dwr == _R + 2:
            mxs = jnp.floor(qxs) <= -1.5
        else:
            mxs = None
        gys = jnp.floor(qys)
        acc0 = jnp.zeros((_TR, _W), jnp.float32)
        acc1 = jnp.zeros((_TR, _W), jnp.float32)
        acc2 = jnp.zeros((_TR, _W), jnp.float32)
        for dh in range(_SPAN):
            dhr = dh - _R
            b = float(0.5 - dhr)
            qx = qxs[dh:dh + _TR]
            qy = qys[dh:dh + _TR]
            nA = nAs[dh:dh + _TR]
            nB = nBs[dh:dh + _TR]
            nC = nCs[dh:dh + _TR]
            fx = a - qx
            fy = b - qy
            power = fx * (nA * fx + nB * fy) + nC * (fy * fy)
            alpha = jnp.exp(jnp.minimum(power, 0.0))
            if dhr == -_R:
                mys = gys[dh:dh + _TR] >= -0.5
            elif dhr == -_R + 1:
                mys = gys[dh:dh + _TR] >= -1.5
            elif dhr == _R + 1:
                mys = gys[dh:dh + _TR] <= -0.5
            elif dhr == _R + 2:
                mys = gys[dh:dh + _TR] <= -1.5
            else:
                mys = None
            if mxs is not None and mys is not None:
                m = mxs[dh:dh + _TR] & mys
            elif mxs is not None:
                m = mxs[dh:dh + _TR]
            else:
                m = mys
            wgt = alpha if m is None else jnp.where(m, alpha, 0.0)
            acc0 = acc0 + wgt * r0s[dh:dh + _TR]
            acc1 = acc1 + wgt * r1s[dh:dh + _TR]
            acc2 = acc2 + wgt * r2s[dh:dh + _TR]
        partials.append((acc0, acc1, acc2))
    while len(partials) > 1:
        nxt = []
        for i in range(0, len(partials) - 1, 2):
            p, q = partials[i], partials[i + 1]
            nxt.append((p[0] + q[0], p[1] + q[1], p[2] + q[2]))
        if len(partials) % 2:
            nxt.append(partials[-1])
        partials = nxt
    tot0, tot1, tot2 = partials[0]
    out_ref[0, 0] = jnp.clip(tot0, 0.0, 1.0)
    out_ref[0, 1] = jnp.clip(tot1, 0.0, 1.0)
    out_ref[0, 2] = jnp.clip(tot2, 0.0, 1.0)


_PARALLEL = pltpu.CompilerParams(dimension_semantics=("parallel", "parallel"))


def kernel(inp, enc_w, enc_b, head_w, head_b):
    w1 = enc_w.reshape(64, 27).astype(jnp.float32)
    w2 = head_w.reshape(8, 64).astype(jnp.float32)
    b1 = enc_b.reshape(64, 1).astype(jnp.float32)
    b2 = head_b.reshape(8, 1).astype(jnp.float32)
    inp_p = jnp.pad(inp.astype(jnp.float32), ((0, 0), (0, 0), (1, 7), (1, 1)))

    params = pl.pallas_call(
        _param_kernel,
        grid=(_B, _NT),
        in_specs=[
            pl.BlockSpec((1, 3, _H + 8, _W + 2), lambda b, t: (b, 0, 0, 0)),
            pl.BlockSpec((64, 27), lambda b, t: (0, 0)),
            pl.BlockSpec((64, 1), lambda b, t: (0, 0)),
            pl.BlockSpec((8, 64), lambda b, t: (0, 0)),
            pl.BlockSpec((8, 1), lambda b, t: (0, 0)),
        ],
        out_specs=pl.BlockSpec((1, 8, _TR, _W), lambda b, t: (b, 0, t, 0)),
        out_shape=jax.ShapeDtypeStruct((_B, 8, _H, _W), jnp.float32),
        compiler_params=_PARALLEL,
    )(inp_p, w1, b1, w2, b2)

    params_p = jnp.pad(params, ((0, 0), (0, 0), (_R, _R + 8), (_R, _R + 2)))

    img = pl.pallas_call(
        _raster_kernel,
        grid=(_B, _NT),
        in_specs=[
            pl.BlockSpec((1, 8, _H + 16, _W + _SPAN - 1),
                         lambda b, t: (b, 0, 0, 0)),
        ],
        out_specs=pl.BlockSpec((1, 3, _TR, _W), lambda b, t: (b, 0, t, 0)),
        out_shape=jax.ShapeDtypeStruct((_B, 3, _H, _W), jnp.float32),
        compiler_params=_PARALLEL,
    )(params_p)
    return img


# speedup vs baseline: 1.2214x; 1.0484x over previous
import math

import jax
import jax.numpy as jnp
from jax.experimental import pallas as pl
from jax.experimental.pallas import tpu as pltpu

_B, _H, _W = 2, 224, 224
_TR = 112
_NT = _H // _TR
_R = 4
_SPAN = 2 * _R + 3
_HR = _TR + _SPAN - 1


def _fused_kernel(inp_ref, w1_ref, b1_ref, w2_ref, b2_ref, out_ref):
    t = pl.program_id(1)
    y0 = t * _TR
    slab = inp_ref[0, :, pl.ds(y0, 128), :]
    cols = []
    for ci in range(3):
        for ky in range(3):
            for kx in range(3):
                cols.append(slab[ci, ky:ky + _HR, kx:kx + _W])
    col = jnp.stack(cols)
    feat = jax.lax.dot_general(
        w1_ref[...], col, (((1,), (0,)), ((), ())),
        preferred_element_type=jnp.float32)
    feat = jnp.maximum(feat + b1_ref[...][:, :, None], 0.0)
    pred = jax.lax.dot_general(
        w2_ref[...], feat, (((1,), (0,)), ((), ())),
        preferred_element_type=jnp.float32)
    pred = pred + b2_ref[...][:, :, None]

    theta = jax.nn.sigmoid(pred[3]) * (2.0 * math.pi)
    ct = jnp.cos(theta)
    st = jnp.sin(theta)
    sx = (jax.nn.sigmoid(pred[4]) * 0.5 + 1e-6) * (_W * 0.5)
    sy = (jax.nn.sigmoid(pred[5]) * 0.5 + 1e-6) * (_H * 0.5)
    sx2 = sx * sx
    sy2 = sy * sy
    ca = ct * ct * sx2 + st * st * sy2
    bcov = ct * st * (sx2 - sy2)
    cc = st * st * sx2 + ct * ct * sy2
    det = ca * cc - bcov * bcov + 1e-12
    inv_det = 1.0 / det

    h_idx = jax.lax.broadcasted_iota(jnp.int32, (_HR, _W), 0) + (y0 - _R)
    in_img = (h_idx >= 0) & (h_idx < _H)
    zero = jnp.zeros((_HR, _W), jnp.float32)

    def xpad(p):
        return jnp.pad(p, ((0, 0), (_R, _R + 2)))

    r0s = xpad(jnp.where(in_img, pred[0], zero))
    r1s = xpad(jnp.where(in_img, pred[1], zero))
    r2s = xpad(jnp.where(in_img, pred[2], zero))
    qxs = xpad(jnp.tanh(pred[6]) - 0.5)
    qys = xpad(jnp.tanh(pred[7]) - 0.5)
    nAs = xpad(-0.5 * cc * inv_det)
    nBs = xpad(bcov * inv_det)
    nCs = xpad(-0.5 * ca * inv_det)

    partials = []
    for dw in range(_SPAN):
        dwr = dw - _R
        a = float(0.5 - dwr)
        r0 = r0s[:, dw:dw + _W]
        r1 = r1s[:, dw:dw + _W]
        r2 = r2s[:, dw:dw + _W]
        qxw = qxs[:, dw:dw + _W]
        qyw = qys[:, dw:dw + _W]
        nAw = nAs[:, dw:dw + _W]
        nBw = nBs[:, dw:dw + _W]
        nCw = nCs[:, dw:dw + _W]
        if dwr == -_R:
            mxs = jnp.floor(qxw) >= -0.5
        elif dwr == -_R + 1:
            mxs = jnp.floor(qxw) >= -1.5
        elif dwr == _R + 1:
            mxs = jnp.floor(qxw) <= -0.5
        elif dwr == _R + 2:
            mxs = jnp.floor(qxw) <= -1.5
        else:
            mxs = None
        gys = jnp.floor(qyw)
        acc0 = jnp.zeros((_TR, _W), jnp.float32)
        acc1 = jnp.zeros((_TR, _W), jnp.float32)
        acc2 = jnp.zeros((_TR, _W), jnp.float32)
        for dh in range(_SPAN):
            dhr = dh - _R
            b = float(0.5 - dhr)
            qx = qxw[dh:dh + _TR]
            qy = qyw[dh:dh + _TR]
            nA = nAw[dh:dh + _TR]
            nB = nBw[dh:dh + _TR]
            nC = nCw[dh:dh + _TR]
            fx = a - qx
            fy = b - qy
            power = fx * (nA * fx + nB * fy) + nC * (fy * fy)
            alpha = jnp.exp(jnp.minimum(power, 0.0))
            if dhr == -_R:
                mys = gys[dh:dh + _TR] >= -0.5
            elif dhr == -_R + 1:
                mys = gys[dh:dh + _TR] >= -1.5
            elif dhr == _R + 1:
                mys = gys[dh:dh + _TR] <= -0.5
            elif dhr == _R + 2:
                mys = gys[dh:dh + _TR] <= -1.5
            else:
                mys = None
            if mxs is not None and mys is not None:
                m = mxs[dh:dh + _TR] & mys
            elif mxs is not None:
                m = mxs[dh:dh + _TR]
            else:
                m = mys
            wgt = alpha if m is None else jnp.where(m, alpha, 0.0)
            acc0 = acc0 + wgt * r0[dh:dh + _TR]
            acc1 = acc1 + wgt * r1[dh:dh + _TR]
            acc2 = acc2 + wgt * r2[dh:dh + _TR]
        partials.append((acc0, acc1, acc2))
    while len(partials) > 1:
        nxt = []
        for i in range(0, len(partials) - 1, 2):
            p, q = partials[i], partials[i + 1]
            nxt.append((p[0] + q[0], p[1] + q[1], p[2] + q[2]))
        if len(partials) % 2:
            nxt.append(partials[-1])
        partials = nxt
    tot0, tot1, tot2 = partials[0]
    out_ref[0, 0] = jnp.clip(tot0, 0.0, 1.0)
    out_ref[0, 1] = jnp.clip(tot1, 0.0, 1.0)
    out_ref[0, 2] = jnp.clip(tot2, 0.0, 1.0)


def kernel(inp, enc_w, enc_b, head_w, head_b):
    w1 = enc_w.reshape(64, 27).astype(jnp.float32)
    w2 = head_w.reshape(8, 64).astype(jnp.float32)
    b1 = enc_b.reshape(64, 1).astype(jnp.float32)
    b2 = head_b.reshape(8, 1).astype(jnp.float32)
    inp_p = jnp.pad(inp.astype(jnp.float32), ((0, 0), (0, 0), (5, 11), (1, 1)))

    img = pl.pallas_call(
        _fused_kernel,
        grid=(_B, _NT),
        in_specs=[
            pl.BlockSpec((1, 3, _H + 16, _W + 2), lambda b, t: (b, 0, 0, 0)),
            pl.BlockSpec((64, 27), lambda b, t: (0, 0)),
            pl.BlockSpec((64, 1), lambda b, t: (0, 0)),
            pl.BlockSpec((8, 64), lambda b, t: (0, 0)),
            pl.BlockSpec((8, 1), lambda b, t: (0, 0)),
        ],
        out_specs=pl.BlockSpec((1, 3, _TR, _W), lambda b, t: (b, 0, t, 0)),
        out_shape=jax.ShapeDtypeStruct((_B, 3, _H, _W), jnp.float32),
        compiler_params=pltpu.CompilerParams(
            dimension_semantics=("parallel", "parallel")),
    )(inp_p, w1, b1, w2, b2)
    return img
